# baseline (speedup 1.0000x reference)
import functools

import jax
import jax.numpy as jnp
from jax import lax
from jax.experimental import pallas as pl
from jax.experimental.pallas import tpu as pltpu
from jax.experimental.pallas import tpu_sc as plsc

_TABLE_PAD = 1024


@functools.lru_cache(maxsize=None)
def _sc_gather(B: int, num_steps: int):
    info = plsc.get_sparse_core_info()
    nc, ns, L = info.num_cores, info.num_subcores, info.num_lanes
    nw = nc * ns
    b_per_w = B // nw
    mesh = plsc.VectorSubcoreMesh(core_axis_name="c", subcore_axis_name="s")

    @functools.partial(
        pl.kernel,
        mesh=mesh,
        out_type=(
            jax.ShapeDtypeStruct((B,), jnp.float32),
            jax.ShapeDtypeStruct((B,), jnp.float32),
        ),
        scratch_types=[
            pltpu.VMEM((_TABLE_PAD,), jnp.float32),
            pltpu.VMEM((_TABLE_PAD,), jnp.float32),
            pltpu.VMEM((b_per_w,), jnp.int32),
            pltpu.VMEM((b_per_w,), jnp.float32),
            pltpu.VMEM((b_per_w,), jnp.float32),
        ],
        compiler_params=pltpu.CompilerParams(needs_layout_passes=False),
    )
    def gather(t_hbm, sab_hbm, somab_hbm, sa_hbm, sb_hbm,
               sab_v, somab_v, idx_v, sa_v, sb_v):
        wid = lax.axis_index("s") * nc + lax.axis_index("c")
        base = wid * b_per_w
        pltpu.sync_copy(sab_hbm, sab_v)
        pltpu.sync_copy(somab_hbm, somab_v)
        pltpu.sync_copy(t_hbm.at[pl.ds(base, b_per_w)], idx_v)
        for j in range(b_per_w // L):
            idx = idx_v[pl.ds(j * L, L)]
            idx = jnp.minimum(jnp.maximum(idx, 0), num_steps - 1)
            sa_v[pl.ds(j * L, L)] = plsc.load_gather(sab_v, [idx])
            sb_v[pl.ds(j * L, L)] = plsc.load_gather(somab_v, [idx])
        pltpu.sync_copy(sa_v, sa_hbm.at[pl.ds(base, b_per_w)])
        pltpu.sync_copy(sb_v, sb_hbm.at[pl.ds(base, b_per_w)])

    return gather


def _make_tc_body(B: int, F: int, R: int, NBUF: int, SPLIT: int = 8):
    NB = B // R
    RS = R // SPLIT

    def body(sa_ref, sb_ref, x_hbm, n_hbm, o_hbm, xb, nb, ob, sx, sn, so):
        def in_copies(s, slot):
            row = slot * R
            for p in range(SPLIT):
                pltpu.make_async_copy(
                    x_hbm.at[pl.ds(s * R + p * RS, RS), :],
                    xb.at[pl.ds(row + p * RS, RS), :], sx.at[slot]).start()
                pltpu.make_async_copy(
                    n_hbm.at[pl.ds(s * R + p * RS, RS), :],
                    nb.at[pl.ds(row + p * RS, RS), :], sn.at[slot]).start()

        for s0 in range(NBUF):
            in_copies(s0, s0)

        def step(s, carry):
            slot = lax.rem(s, NBUF)
            row = slot * R
            pltpu.make_async_copy(
                x_hbm.at[pl.ds(s * R, R), :], xb.at[pl.ds(row, R), :],
                sx.at[slot]).wait()
            pltpu.make_async_copy(
                n_hbm.at[pl.ds(s * R, R), :], nb.at[pl.ds(row, R), :],
                sn.at[slot]).wait()

            @pl.when(s >= NBUF)
            def _():
                pltpu.make_async_copy(
                    ob.at[pl.ds(row, R), :],
                    o_hbm.at[pl.ds((s - NBUF) * R, R), :], so.at[slot]).wait()

            sa = sa_ref[pl.ds(s * R, R), :]
            sb = sb_ref[pl.ds(s * R, R), :]
            ob[pl.ds(row, R), :] = (sa * xb[pl.ds(row, R), :]
                                    + sb * nb[pl.ds(row, R), :])
            for p in range(SPLIT):
                pltpu.make_async_copy(
                    ob.at[pl.ds(row + p * RS, RS), :],
                    o_hbm.at[pl.ds(s * R + p * RS, RS), :],
                    so.at[slot]).start()

            @pl.when(s + NBUF < NB)
            def _():
                in_copies(s + NBUF, lax.rem(s + NBUF, NBUF))

            return carry

        lax.fori_loop(0, NB, step, 0)
        for k in range(NBUF):
            s = NB - NBUF + k
            slot = s % NBUF
            pltpu.make_async_copy(
                ob.at[pl.ds(slot * R, R), :], o_hbm.at[pl.ds(s * R, R), :],
                so.at[slot]).wait()

    return body


@functools.lru_cache(maxsize=None)
def _tc_combine(B: int, F: int, R: int, NBUF: int = 3):
    shape = jax.ShapeDtypeStruct((B, F), jnp.float32)
    return pl.pallas_call(
        _make_tc_body(B, F, R, NBUF),
        in_specs=[
            pl.BlockSpec(memory_space=pltpu.VMEM),
            pl.BlockSpec(memory_space=pltpu.VMEM),
            pl.BlockSpec(memory_space=pl.ANY),
            pl.BlockSpec(memory_space=pl.ANY),
        ],
        out_specs=pl.BlockSpec(memory_space=pl.ANY),
        out_shape=shape,
        scratch_shapes=[
            pltpu.VMEM((NBUF * R, F), jnp.float32),
            pltpu.VMEM((NBUF * R, F), jnp.float32),
            pltpu.VMEM((NBUF * R, F), jnp.float32),
            pltpu.SemaphoreType.DMA((NBUF,)),
            pltpu.SemaphoreType.DMA((NBUF,)),
            pltpu.SemaphoreType.DMA((NBUF,)),
        ],
    )


def kernel(x_start, noise, t, sqrt_alpha_bars, sqrt_one_minus_alpha_bars):
    B = x_start.shape[0]
    F = x_start.size // B
    num_steps = sqrt_alpha_bars.shape[0]
    pad = _TABLE_PAD - num_steps
    sab = jnp.pad(sqrt_alpha_bars, (0, pad))
    somab = jnp.pad(sqrt_one_minus_alpha_bars, (0, pad))
    sa, sb = _sc_gather(B, num_steps)(t, sab, somab)
    x_t = _tc_combine(B, F, 64)(
        sa.reshape(B, 1), sb.reshape(B, 1),
        x_start.reshape(B, F), noise.reshape(B, F))
    return (x_t.reshape(x_start.shape), noise)


# speedup vs baseline: 2.3891x; 2.3891x over previous
import functools

import jax
import jax.numpy as jnp
from jax import lax
from jax.experimental import pallas as pl
from jax.experimental.pallas import tpu as pltpu
from jax.experimental.pallas import tpu_sc as plsc

_TABLE_PAD = 1024


@functools.lru_cache(maxsize=None)
def _sc_gather(B: int, num_steps: int):
    info = plsc.get_sparse_core_info()
    nc, ns, L = info.num_cores, info.num_subcores, info.num_lanes
    nw = nc * ns
    b_per_w = B // nw
    mesh = plsc.VectorSubcoreMesh(core_axis_name="c", subcore_axis_name="s")

    @functools.partial(
        pl.kernel,
        mesh=mesh,
        out_type=(
            jax.ShapeDtypeStruct((B,), jnp.float32),
            jax.ShapeDtypeStruct((B,), jnp.float32),
        ),
        scratch_types=[
            pltpu.VMEM((_TABLE_PAD,), jnp.float32),
            pltpu.VMEM((_TABLE_PAD,), jnp.float32),
            pltpu.VMEM((b_per_w,), jnp.int32),
            pltpu.VMEM((b_per_w,), jnp.float32),
            pltpu.VMEM((b_per_w,), jnp.float32),
        ],
        compiler_params=pltpu.CompilerParams(needs_layout_passes=False),
    )
    def gather(t_hbm, sab_hbm, somab_hbm, sa_hbm, sb_hbm,
               sab_v, somab_v, idx_v, sa_v, sb_v):
        wid = lax.axis_index("s") * nc + lax.axis_index("c")
        base = wid * b_per_w
        pltpu.sync_copy(sab_hbm, sab_v)
        pltpu.sync_copy(somab_hbm, somab_v)
        pltpu.sync_copy(t_hbm.at[pl.ds(base, b_per_w)], idx_v)
        for j in range(b_per_w // L):
            idx = idx_v[pl.ds(j * L, L)]
            idx = jnp.minimum(jnp.maximum(idx, 0), num_steps - 1)
            sa_v[pl.ds(j * L, L)] = plsc.load_gather(sab_v, [idx])
            sb_v[pl.ds(j * L, L)] = plsc.load_gather(somab_v, [idx])
        pltpu.sync_copy(sa_v, sa_hbm.at[pl.ds(base, b_per_w)])
        pltpu.sync_copy(sb_v, sb_hbm.at[pl.ds(base, b_per_w)])

    return gather


def _tc_combine_body(sa_ref, sb_ref, x_ref, n_ref, o_ref):
    o_ref[...] = sa_ref[...] * x_ref[...] + sb_ref[...] * n_ref[...]


@functools.lru_cache(maxsize=None)
def _tc_combine(F: int, B: int, R: int):
    data = pl.BlockSpec((R, B), lambda i: (i, 0))
    scale = pl.BlockSpec((1, B), lambda i: (0, 0))
    shape = jax.ShapeDtypeStruct((F, B), jnp.float32)
    return pl.pallas_call(
        _tc_combine_body,
        grid=(F // R,),
        in_specs=[scale, scale, data, data],
        out_specs=data,
        out_shape=shape,
    )


def kernel(x_start, noise, t, sqrt_alpha_bars, sqrt_one_minus_alpha_bars):
    B, C, H, W = x_start.shape
    F = C * H * W
    num_steps = sqrt_alpha_bars.shape[0]
    pad = _TABLE_PAD - num_steps
    sab = jnp.pad(sqrt_alpha_bars, (0, pad))
    somab = jnp.pad(sqrt_one_minus_alpha_bars, (0, pad))
    sa, sb = _sc_gather(B, num_steps)(t, sab, somab)
    xT = jnp.transpose(x_start, (1, 2, 3, 0)).reshape(F, B)
    nT = jnp.transpose(noise, (1, 2, 3, 0)).reshape(F, B)
    oT = _tc_combine(F, B, 512)(sa.reshape(1, B), sb.reshape(1, B), xT, nT)
    x_t = jnp.transpose(oT.reshape(C, H, W, B), (3, 0, 1, 2))
    return (x_t, noise)


# speedup vs baseline: 2.4035x; 1.0061x over previous
import functools

import jax
import jax.numpy as jnp
from jax import lax
from jax.experimental import pallas as pl
from jax.experimental.pallas import tpu as pltpu
from jax.experimental.pallas import tpu_sc as plsc

_TABLE_PAD = 1024


@functools.lru_cache(maxsize=None)
def _sc_gather(B: int, num_steps: int):
    info = plsc.get_sparse_core_info()
    nc, ns, L = info.num_cores, info.num_subcores, info.num_lanes
    nw = nc * ns
    b_per_w = B // nw
    mesh = plsc.VectorSubcoreMesh(core_axis_name="c", subcore_axis_name="s")

    @functools.partial(
        pl.kernel,
        mesh=mesh,
        out_type=(
            jax.ShapeDtypeStruct((B,), jnp.float32),
            jax.ShapeDtypeStruct((B,), jnp.float32),
        ),
        scratch_types=[
            pltpu.VMEM((_TABLE_PAD,), jnp.float32),
            pltpu.VMEM((_TABLE_PAD,), jnp.float32),
            pltpu.VMEM((b_per_w,), jnp.int32),
            pltpu.VMEM((b_per_w,), jnp.float32),
            pltpu.VMEM((b_per_w,), jnp.float32),
        ],
        compiler_params=pltpu.CompilerParams(needs_layout_passes=False),
    )
    def gather(t_hbm, sab_hbm, somab_hbm, sa_hbm, sb_hbm,
               sab_v, somab_v, idx_v, sa_v, sb_v):
        wid = lax.axis_index("s") * nc + lax.axis_index("c")
        base = wid * b_per_w
        pltpu.sync_copy(sab_hbm, sab_v)
        pltpu.sync_copy(somab_hbm, somab_v)
        pltpu.sync_copy(t_hbm.at[pl.ds(base, b_per_w)], idx_v)
        for j in range(b_per_w // L):
            idx = idx_v[pl.ds(j * L, L)]
            idx = jnp.minimum(jnp.maximum(idx, 0), num_steps - 1)
            sa_v[pl.ds(j * L, L)] = plsc.load_gather(sab_v, [idx])
            sb_v[pl.ds(j * L, L)] = plsc.load_gather(somab_v, [idx])
        pltpu.sync_copy(sa_v, sa_hbm.at[pl.ds(base, b_per_w)])
        pltpu.sync_copy(sb_v, sb_hbm.at[pl.ds(base, b_per_w)])

    return gather


def _tc_combine_body(sa_ref, sb_ref, x_ref, n_ref, o_ref):
    o_ref[...] = sa_ref[...] * x_ref[...] + sb_ref[...] * n_ref[...]


@functools.lru_cache(maxsize=None)
def _tc_combine(F: int, B: int, R: int):
    data = pl.BlockSpec((R, B), lambda i: (i, 0))
    scale = pl.BlockSpec((1, B), lambda i: (0, 0))
    shape = jax.ShapeDtypeStruct((F, B), jnp.float32)
    return pl.pallas_call(
        _tc_combine_body,
        grid=(F // R,),
        in_specs=[scale, scale, data, data],
        out_specs=data,
        out_shape=shape,
    )


def kernel(x_start, noise, t, sqrt_alpha_bars, sqrt_one_minus_alpha_bars):
    B, C, H, W = x_start.shape
    F = C * H * W
    num_steps = sqrt_alpha_bars.shape[0]
    pad = _TABLE_PAD - num_steps
    sab = jnp.pad(sqrt_alpha_bars, (0, pad))
    somab = jnp.pad(sqrt_one_minus_alpha_bars, (0, pad))
    sa, sb = _sc_gather(B, num_steps)(t, sab, somab)
    xT = jnp.transpose(x_start, (1, 2, 3, 0)).reshape(F, B)
    nT = jnp.transpose(noise, (1, 2, 3, 0)).reshape(F, B)
    oT = _tc_combine(F, B, 1024)(sa.reshape(1, B), sb.reshape(1, B), xT, nT)
    x_t = jnp.transpose(oT.reshape(C, H, W, B), (3, 0, 1, 2))
    return (x_t, noise)


# speedup vs baseline: 2.4083x; 1.0020x over previous
import functools

import jax
import jax.numpy as jnp
from jax import lax
from jax.experimental import pallas as pl
from jax.experimental.pallas import tpu as pltpu
from jax.experimental.pallas import tpu_sc as plsc

_TABLE_PAD = 1024


@functools.lru_cache(maxsize=None)
def _sc_gather(B: int, num_steps: int):
    info = plsc.get_sparse_core_info()
    nc, ns, L = info.num_cores, info.num_subcores, info.num_lanes
    nw = nc * ns
    b_per_w = B // nw
    mesh = plsc.VectorSubcoreMesh(core_axis_name="c", subcore_axis_name="s")

    @functools.partial(
        pl.kernel,
        mesh=mesh,
        out_type=jax.ShapeDtypeStruct((2, B), jnp.float32),
        scratch_types=[
            pltpu.VMEM((2 * _TABLE_PAD,), jnp.float32),
            pltpu.VMEM((b_per_w,), jnp.int32),
            pltpu.VMEM((b_per_w,), jnp.float32),
            pltpu.VMEM((b_per_w,), jnp.float32),
            pltpu.SemaphoreType.DMA,
            pltpu.SemaphoreType.DMA,
        ],
        compiler_params=pltpu.CompilerParams(needs_layout_passes=False),
    )
    def gather(t_hbm, tab_hbm, out_hbm, tab_v, idx_v, sa_v, sb_v, s0, s1):
        wid = lax.axis_index("s") * nc + lax.axis_index("c")
        base = wid * b_per_w
        ctab = pltpu.make_async_copy(tab_hbm, tab_v, s0)
        cidx = pltpu.make_async_copy(t_hbm.at[pl.ds(base, b_per_w)], idx_v, s1)
        ctab.start()
        cidx.start()
        ctab.wait()
        cidx.wait()
        for j in range(b_per_w // L):
            idx = idx_v[pl.ds(j * L, L)]
            idx = jnp.minimum(jnp.maximum(idx, 0), num_steps - 1)
            sa_v[pl.ds(j * L, L)] = plsc.load_gather(tab_v, [idx])
            sb_v[pl.ds(j * L, L)] = plsc.load_gather(tab_v, [idx + _TABLE_PAD])
        ca = pltpu.make_async_copy(sa_v, out_hbm.at[0, pl.ds(base, b_per_w)], s0)
        cb = pltpu.make_async_copy(sb_v, out_hbm.at[1, pl.ds(base, b_per_w)], s1)
        ca.start()
        cb.start()
        ca.wait()
        cb.wait()

    return gather


def _tc_combine_body(sc_ref, x_ref, n_ref, o_ref):
    o_ref[...] = (sc_ref[0:1, :] * x_ref[...]
                  + sc_ref[1:2, :] * n_ref[...])


@functools.lru_cache(maxsize=None)
def _tc_combine(F: int, B: int, R: int):
    data = pl.BlockSpec((R, B), lambda i: (i, 0))
    scale = pl.BlockSpec((2, B), lambda i: (0, 0))
    shape = jax.ShapeDtypeStruct((F, B), jnp.float32)
    return pl.pallas_call(
        _tc_combine_body,
        grid=(F // R,),
        in_specs=[scale, data, data],
        out_specs=data,
        out_shape=shape,
    )


def kernel(x_start, noise, t, sqrt_alpha_bars, sqrt_one_minus_alpha_bars):
    B, C, H, W = x_start.shape
    F = C * H * W
    num_steps = sqrt_alpha_bars.shape[0]
    pad = _TABLE_PAD - num_steps
    tab = jnp.concatenate([
        jnp.pad(sqrt_alpha_bars, (0, pad)),
        jnp.pad(sqrt_one_minus_alpha_bars, (0, pad)),
    ])
    scales = _sc_gather(B, num_steps)(t, tab)
    xT = jnp.transpose(x_start, (1, 2, 3, 0)).reshape(F, B)
    nT = jnp.transpose(noise, (1, 2, 3, 0)).reshape(F, B)
    oT = _tc_combine(F, B, 1024)(scales, xT, nT)
    x_t = jnp.transpose(oT.reshape(C, H, W, B), (3, 0, 1, 2))
    return (x_t, noise)
